# baseline (speedup 1.0000x reference)
import jax
import jax.numpy as jnp
from jax.experimental import pallas as pl
from jax.experimental.pallas import tpu as pltpu


def _lstm_head_body(h_ref, wih_ref, b_ref, wfc_ref, bfc_ref, o_ref):
    h = h_ref[...]
    gates = jnp.dot(h, wih_ref[...].T, preferred_element_type=jnp.float32)
    gates = gates + b_ref[...]
    i, f, g, o = jnp.split(gates, 4, axis=1)
    c = jax.nn.sigmoid(i) * jnp.tanh(g)
    hd = jax.nn.sigmoid(o) * jnp.tanh(c)
    o_ref[...] = jnp.sum(hd * wfc_ref[...], axis=1, keepdims=True) + bfc_ref[0, 0]


def _lstm_head(h, Wih, bih, bhh, Wfc, bfc):
    N = h.shape[0]
    BN = 2000
    b = (bih + bhh).reshape(1, 128)
    return pl.pallas_call(
        _lstm_head_body,
        grid=(N // BN,),
        in_specs=[
            pl.BlockSpec((BN, 128), lambda i: (i, 0)),
            pl.BlockSpec((128, 128), lambda i: (0, 0)),
            pl.BlockSpec((1, 128), lambda i: (0, 0)),
            pl.BlockSpec((1, 32), lambda i: (0, 0)),
            pl.BlockSpec((1, 1), lambda i: (0, 0)),
        ],
        out_specs=pl.BlockSpec((BN, 1), lambda i: (i, 0)),
        out_shape=jax.ShapeDtypeStruct((N, 1), jnp.float32),
    )(h, Wih, b, Wfc, bfc.reshape(1, 1))


def _gatv2(x, src, dst, ea, Wl, bl, Wr, br, We, att, bias, heads, out_ch, concat, N):
    xl = (x @ Wl.T + bl).reshape(N, heads, out_ch)
    xr = (x @ Wr.T + br).reshape(N, heads, out_ch)
    ee = (ea @ We.T).reshape(-1, heads, out_ch)
    m = jax.nn.leaky_relu(xl[src] + xr[dst] + ee, 0.2)
    alpha = (m * att).sum(-1)
    amax = jax.ops.segment_max(alpha, dst, num_segments=N)
    ex = jnp.exp(alpha - amax[dst])
    den = jax.ops.segment_sum(ex, dst, num_segments=N)
    a = ex / den[dst]
    out = jax.ops.segment_sum(xl[src] * a[:, :, None], dst, num_segments=N)
    if concat:
        return out.reshape(N, heads * out_ch) + bias
    return out.mean(axis=1) + bias


def kernel(x, edge_index, edge_attr, Wl1, bl1, Wr1, br1, We1, att1, bias1,
           Wl2, bl2, Wr2, br2, We2, att2, bias2, Wih, Whh, bih, bhh, Wfc, bfc):
    N = x.shape[0]
    dst0 = edge_index[1]
    ones = jnp.ones((edge_attr.shape[0],), dtype=edge_attr.dtype)
    cnt = jax.ops.segment_sum(ones, dst0, num_segments=N)
    mean_attr = jax.ops.segment_sum(edge_attr, dst0, num_segments=N) / jnp.maximum(cnt, 1.0)[:, None]
    ar = jnp.arange(N, dtype=edge_index.dtype)
    src = jnp.concatenate([edge_index[0], ar])
    dst = jnp.concatenate([dst0, ar])
    ea = jnp.concatenate([edge_attr, mean_attr], axis=0)

    h = jnp.tanh(_gatv2(x, src, dst, ea, Wl1, bl1, Wr1, br1, We1, att1, bias1, 8, 64, True, N))
    h = jnp.tanh(_gatv2(h, src, dst, ea, Wl2, bl2, Wr2, br2, We2, att2, bias2, 8, 128, False, N))
    return _lstm_head(h, Wih, bih, bhh, Wfc, bfc)


# speedup vs baseline: 6.6837x; 6.6837x over previous
import functools

import jax
import jax.numpy as jnp
from jax import lax
from jax.experimental import pallas as pl
from jax.experimental.pallas import tpu as pltpu
from jax.experimental.pallas import tpu_sc as plsc

N_NODES = 10000
N_PAD = 10112
N_EDGES = 160000
E0_PAD = 163840
EP = 170000
EP_PAD = 172032
NW = 32
_MESH = dict(core_axis_name="c", subcore_axis_name="s")


def _sc_gather(table, idx, block_rows):
    n_rows, d = table.shape
    b = idx.shape[0]
    bpw = b // NW
    nblk = bpw // block_rows

    @functools.partial(
        pl.kernel,
        mesh=plsc.VectorSubcoreMesh(**_MESH),
        out_type=jax.ShapeDtypeStruct((b, d), jnp.float32),
        scratch_types=[
            pltpu.VMEM((block_rows,), jnp.int32),
            pltpu.VMEM((block_rows, d), jnp.float32),
            pltpu.SemaphoreType.DMA,
        ],
    )
    def k(table_hbm, idx_hbm, out_hbm, idx_v, rows_v, sem):
        wid = lax.axis_index("s") * 2 + lax.axis_index("c")
        base = wid * bpw

        def body(i, _):
            off = base + i * block_rows
            pltpu.sync_copy(idx_hbm.at[pl.ds(off, block_rows)], idx_v)
            pltpu.async_copy(table_hbm.at[idx_v], rows_v, sem).wait()
            pltpu.sync_copy(rows_v, out_hbm.at[pl.ds(off, block_rows)])
            return 0

        lax.fori_loop(0, nblk, body, 0)

    return k(table, idx)


def _sc_scatter_add(rows, idx, n_chunks, block_rows):
    b, d = rows.shape
    dc = d // n_chunks
    bpw = b // NW
    nblk = bpw // block_rows
    rpt = N_PAD // 16
    zeros_pad = jnp.zeros((N_PAD, dc), jnp.float32)

    @functools.partial(
        pl.kernel,
        mesh=plsc.VectorSubcoreMesh(**_MESH),
        out_type=jax.ShapeDtypeStruct((2, N_PAD, d), jnp.float32),
        scratch_types=[
            pltpu.VMEM((block_rows,), jnp.int32),
            pltpu.VMEM((block_rows, dc), jnp.float32),
            pltpu.VMEM_SHARED((N_PAD, dc), jnp.float32),
        ],
    )
    def k(rows_hbm, idx_hbm, zero_hbm, out_hbm, idx_v, rows_v, acc):
        cid = lax.axis_index("c")
        sid = lax.axis_index("s")
        wid = sid * 2 + cid
        base = wid * bpw
        r0 = sid * rpt

        for ch in range(n_chunks):
            c0 = ch * dc
            pltpu.sync_copy(zero_hbm.at[pl.ds(r0, rpt)], acc.at[pl.ds(r0, rpt)])
            plsc.subcore_barrier()

            def body(i, _):
                off = base + i * block_rows
                pltpu.sync_copy(idx_hbm.at[pl.ds(off, block_rows)], idx_v)
                if n_chunks == 1:
                    pltpu.sync_copy(rows_hbm.at[pl.ds(off, block_rows)], rows_v)
                else:
                    pltpu.sync_copy(
                        rows_hbm.at[pl.ds(off, block_rows), pl.ds(c0, dc)],
                        rows_v)
                pltpu.sync_copy(rows_v, acc.at[idx_v], add=True)
                return 0

            lax.fori_loop(0, nblk, body, 0)
            plsc.subcore_barrier()
            if n_chunks == 1:
                pltpu.sync_copy(acc.at[pl.ds(r0, rpt)],
                                out_hbm.at[cid, pl.ds(r0, rpt)])
            else:
                pltpu.sync_copy(
                    acc.at[pl.ds(r0, rpt)],
                    out_hbm.at[cid, pl.ds(r0, rpt), pl.ds(c0, dc)])

    return k(rows, idx, zeros_pad)


def _proj_body(x_ref, wl_ref, wr_ref, bl_ref, br_ref, xl_ref, xr_ref):
    xv = x_ref[...]
    xl_ref[...] = jnp.dot(xv, wl_ref[...], preferred_element_type=jnp.float32) + bl_ref[...]
    xr_ref[...] = jnp.dot(xv, wr_ref[...], preferred_element_type=jnp.float32) + br_ref[...]


def _proj(xin, wlT, wrT, bl, br, bn):
    n, kdim = xin.shape
    d = wlT.shape[1]
    out = jax.ShapeDtypeStruct((n, d), jnp.float32)
    return pl.pallas_call(
        _proj_body,
        grid=(n // bn,),
        in_specs=[
            pl.BlockSpec((bn, kdim), lambda i: (i, 0)),
            pl.BlockSpec((kdim, d), lambda i: (0, 0)),
            pl.BlockSpec((kdim, d), lambda i: (0, 0)),
            pl.BlockSpec((1, d), lambda i: (0, 0)),
            pl.BlockSpec((1, d), lambda i: (0, 0)),
        ],
        out_specs=[
            pl.BlockSpec((bn, d), lambda i: (i, 0)),
            pl.BlockSpec((bn, d), lambda i: (i, 0)),
        ],
        out_shape=[out, out],
    )(xin, wlT, wrT, bl.reshape(1, d), br.reshape(1, d))


def _gat_proj_body(p_ref, bias_ref, wl_ref, wr_ref, bl_ref, br_ref, xl_ref, xr_ref):
    h = jnp.tanh(p_ref[0] + p_ref[1] + bias_ref[...])
    xl_ref[...] = jnp.dot(h, wl_ref[...], preferred_element_type=jnp.float32) + bl_ref[...]
    xr_ref[...] = jnp.dot(h, wr_ref[...], preferred_element_type=jnp.float32) + br_ref[...]


def _gat_proj(parts, bias, wlT, wrT, bl, br, bn):
    _, n, kdim = parts.shape
    d = wlT.shape[1]
    out = jax.ShapeDtypeStruct((n, d), jnp.float32)
    return pl.pallas_call(
        _gat_proj_body,
        grid=(n // bn,),
        in_specs=[
            pl.BlockSpec((2, bn, kdim), lambda i: (0, i, 0)),
            pl.BlockSpec((1, kdim), lambda i: (0, 0)),
            pl.BlockSpec((kdim, d), lambda i: (0, 0)),
            pl.BlockSpec((kdim, d), lambda i: (0, 0)),
            pl.BlockSpec((1, d), lambda i: (0, 0)),
            pl.BlockSpec((1, d), lambda i: (0, 0)),
        ],
        out_specs=[
            pl.BlockSpec((bn, d), lambda i: (i, 0)),
            pl.BlockSpec((bn, d), lambda i: (i, 0)),
        ],
        out_shape=[out, out],
    )(parts, bias.reshape(1, kdim), wlT, wrT, bl.reshape(1, d), br.reshape(1, d))


def _alpha_body(xlg_ref, xrg_ref, ea_ref, we_ref, att_ref, sel_ref, o_ref):
    u = xlg_ref[...] + xrg_ref[...] + jnp.dot(
        ea_ref[...], we_ref[...], preferred_element_type=jnp.float32)
    m = jnp.maximum(u, 0.2 * u)
    o_ref[...] = jnp.dot(m * att_ref[...], sel_ref[...],
                         preferred_element_type=jnp.float32)


def _alpha(xlg, xrg, ea, weT, att_row, sel, be):
    b, d = xlg.shape
    return pl.pallas_call(
        _alpha_body,
        grid=(b // be,),
        in_specs=[
            pl.BlockSpec((be, d), lambda i: (i, 0)),
            pl.BlockSpec((be, d), lambda i: (i, 0)),
            pl.BlockSpec((be, 16), lambda i: (i, 0)),
            pl.BlockSpec((16, d), lambda i: (0, 0)),
            pl.BlockSpec((1, d), lambda i: (0, 0)),
            pl.BlockSpec((d, 128), lambda i: (0, 0)),
        ],
        out_specs=pl.BlockSpec((be, 128), lambda i: (i, 0)),
        out_shape=jax.ShapeDtypeStruct((b, 128), jnp.float32),
    )(xlg, xrg, ea, weT, att_row, sel)


def _exp_body(a_ref, g_ref, o_ref, *, be, n_valid):
    gid = pl.program_id(0) * be + lax.broadcasted_iota(jnp.int32, (be, 128), 0)
    lane = lax.broadcasted_iota(jnp.int32, (be, 128), 1)
    ok = (gid < n_valid) & (lane < 8)
    o_ref[...] = jnp.where(ok, jnp.exp(a_ref[...] - g_ref[0, 0]), 0.0)


def _exp_mask(alpha, gmax, be):
    b = alpha.shape[0]
    return pl.pallas_call(
        functools.partial(_exp_body, be=be, n_valid=EP),
        grid=(b // be,),
        in_specs=[
            pl.BlockSpec((be, 128), lambda i: (i, 0)),
            pl.BlockSpec((1, 1), lambda i: (0, 0)),
        ],
        out_specs=pl.BlockSpec((be, 128), lambda i: (i, 0)),
        out_shape=jax.ShapeDtypeStruct((b, 128), jnp.float32),
    )(alpha, gmax.reshape(1, 1))


def _weight_body(xlg_ref, ex_ref, dg_ref, bsel_ref, o_ref):
    dg = dg_ref[...]
    w16 = ex_ref[...] / jnp.where(dg > 0.0, dg, 1.0)
    wfull = jnp.dot(w16, bsel_ref[...], preferred_element_type=jnp.float32)
    o_ref[...] = wfull * xlg_ref[...]


def _weight(xlg, ex, deng, bsel, be):
    b, d = xlg.shape
    return pl.pallas_call(
        _weight_body,
        grid=(b // be,),
        in_specs=[
            pl.BlockSpec((be, d), lambda i: (i, 0)),
            pl.BlockSpec((be, 128), lambda i: (i, 0)),
            pl.BlockSpec((be, 128), lambda i: (i, 0)),
            pl.BlockSpec((128, d), lambda i: (0, 0)),
        ],
        out_specs=pl.BlockSpec((be, d), lambda i: (i, 0)),
        out_shape=jax.ShapeDtypeStruct((b, d), jnp.float32),
    )(xlg, ex, deng, bsel)


def _head_body(p_ref, bias2_ref, wih_ref, b_ref, wfc_ref, bfc_ref, o_ref):
    s = p_ref[0] + p_ref[1]
    mean = s[:, 0:128]
    for h in range(1, 8):
        mean = mean + s[:, h * 128:(h + 1) * 128]
    h2 = jnp.tanh(mean * 0.125 + bias2_ref[...])
    gates = jnp.dot(h2, wih_ref[...], preferred_element_type=jnp.float32) + b_ref[...]
    i, f, g, o = jnp.split(gates, 4, axis=1)
    c = jax.nn.sigmoid(i) * jnp.tanh(g)
    hd = jax.nn.sigmoid(o) * jnp.tanh(c)
    o_ref[...] = jnp.sum(hd * wfc_ref[...], axis=1, keepdims=True) + bfc_ref[0, 0]


def _head(parts, bias2, WihT, b, Wfc, bfc, bn):
    _, n, d = parts.shape
    return pl.pallas_call(
        _head_body,
        grid=(n // bn,),
        in_specs=[
            pl.BlockSpec((2, bn, d), lambda i: (0, i, 0)),
            pl.BlockSpec((1, 128), lambda i: (0, 0)),
            pl.BlockSpec((128, 128), lambda i: (0, 0)),
            pl.BlockSpec((1, 128), lambda i: (0, 0)),
            pl.BlockSpec((1, 32), lambda i: (0, 0)),
            pl.BlockSpec((1, 1), lambda i: (0, 0)),
        ],
        out_specs=pl.BlockSpec((bn, 1), lambda i: (i, 0)),
        out_shape=jax.ShapeDtypeStruct((n, 1), jnp.float32),
    )(parts, bias2.reshape(1, 128), WihT, b.reshape(1, 128), Wfc, bfc.reshape(1, 1))


def _selectors(d, c_per_head):
    ch = jnp.arange(d, dtype=jnp.int32) // c_per_head
    hh = jnp.arange(128, dtype=jnp.int32)
    sel = ((ch[:, None] == hh[None, :]) & (hh[None, :] < 8)).astype(jnp.float32)
    return sel, sel.T


def _pad_rows(a, n):
    return jnp.pad(a, ((0, n - a.shape[0]),) + ((0, 0),) * (a.ndim - 1))


def kernel(x, edge_index, edge_attr, Wl1, bl1, Wr1, br1, We1, att1, bias1,
           Wl2, bl2, Wr2, br2, We2, att2, bias2, Wih, Whh, bih, bhh, Wfc, bfc):
    n = x.shape[0]
    src0 = edge_index[0].astype(jnp.int32)
    dst0 = edge_index[1].astype(jnp.int32)
    ar = jnp.arange(n, dtype=jnp.int32)
    src_p = _pad_rows(jnp.concatenate([src0, ar])[:, None], EP_PAD)[:, 0]
    dst_p = _pad_rows(jnp.concatenate([dst0, ar])[:, None], EP_PAD)[:, 0]

    ea_ext = jnp.concatenate(
        [edge_attr, jnp.ones((N_EDGES, 1), jnp.float32),
         jnp.zeros((N_EDGES, 111), jnp.float32)], axis=1)
    s0 = _sc_scatter_add(_pad_rows(ea_ext, E0_PAD),
                         _pad_rows(dst0[:, None], E0_PAD)[:, 0],
                         n_chunks=1, block_rows=128)
    ssum = s0[0] + s0[1]
    cnt = ssum[:n, 16:17]
    mean_attr = ssum[:n, :16] / jnp.maximum(cnt, 1.0)
    ea_full = _pad_rows(jnp.concatenate([edge_attr, mean_attr], axis=0), EP_PAD)

    x_p = _pad_rows(x, N_PAD)

    sel1, bsel1 = _selectors(512, 64)
    xl1, xr1 = _proj(x_p, Wl1.T, Wr1.T, bl1, br1, bn=632)
    xlg1 = _sc_gather(xl1, src_p, block_rows=64)
    xrg1 = _sc_gather(xr1, dst_p, block_rows=64)
    alpha1 = _alpha(xlg1, xrg1, ea_full, We1.T, att1.reshape(1, 512), sel1, be=2048)
    gmax1 = jnp.max(alpha1[:, :8])
    ex1 = _exp_mask(alpha1, gmax1, be=2048)
    den1p = _sc_scatter_add(ex1, dst_p, n_chunks=1, block_rows=128)
    deng1 = _sc_gather(den1p[0] + den1p[1], dst_p, block_rows=128)
    w1 = _weight(xlg1, ex1, deng1, bsel1, be=2048)
    out1p = _sc_scatter_add(w1, dst_p, n_chunks=4, block_rows=64)

    sel2, bsel2 = _selectors(1024, 128)
    xl2, xr2 = _gat_proj(out1p, bias1, Wl2.T, Wr2.T, bl2, br2, bn=632)
    xlg2 = _sc_gather(xl2, src_p, block_rows=32)
    xrg2 = _sc_gather(xr2, dst_p, block_rows=32)
    alpha2 = _alpha(xlg2, xrg2, ea_full, We2.T, att2.reshape(1, 1024), sel2, be=1024)
    gmax2 = jnp.max(alpha2[:, :8])
    ex2 = _exp_mask(alpha2, gmax2, be=2048)
    den2p = _sc_scatter_add(ex2, dst_p, n_chunks=1, block_rows=128)
    deng2 = _sc_gather(den2p[0] + den2p[1], dst_p, block_rows=128)
    w2 = _weight(xlg2, ex2, deng2, bsel2, be=1024)
    out2p = _sc_scatter_add(w2, dst_p, n_chunks=8, block_rows=64)

    y = _head(out2p, bias2, Wih.T, bih + bhh, Wfc, bfc, bn=632)
    return y[:n]


# speedup vs baseline: 7.3537x; 1.1002x over previous
import functools

import jax
import jax.numpy as jnp
from jax import lax
from jax.experimental import pallas as pl
from jax.experimental.pallas import tpu as pltpu
from jax.experimental.pallas import tpu_sc as plsc

N_NODES = 10000
N_PAD = 10112
N_EDGES = 160000
E0_PAD = 163840
EP = 170000
EP_PAD = 172032
NW = 32
_MESH = dict(core_axis_name="c", subcore_axis_name="s")


def _sc_gather(table, idx, block_rows):
    n_rows, d = table.shape
    b = idx.shape[0]
    bpw = b // NW
    nblk = bpw // block_rows

    assert nblk % 2 == 0

    @functools.partial(
        pl.kernel,
        mesh=plsc.VectorSubcoreMesh(**_MESH),
        out_type=jax.ShapeDtypeStruct((b, d), jnp.float32),
        scratch_types=[
            pltpu.VMEM((block_rows,), jnp.int32),
            pltpu.VMEM((block_rows,), jnp.int32),
            pltpu.VMEM((block_rows, d), jnp.float32),
            pltpu.VMEM((block_rows, d), jnp.float32),
            pltpu.SemaphoreType.DMA,
            pltpu.SemaphoreType.DMA,
        ],
    )
    def k(table_hbm, idx_hbm, out_hbm, idx0, idx1, rows0, rows1, sem0, sem1):
        wid = lax.axis_index("s") * 2 + lax.axis_index("c")
        base = wid * bpw

        def body(j, _):
            off0 = base + (2 * j) * block_rows
            off1 = off0 + block_rows
            pltpu.sync_copy(idx_hbm.at[pl.ds(off0, block_rows)], idx0)
            pltpu.sync_copy(idx_hbm.at[pl.ds(off1, block_rows)], idx1)
            g0 = pltpu.async_copy(table_hbm.at[idx0], rows0, sem0)
            g1 = pltpu.async_copy(table_hbm.at[idx1], rows1, sem1)
            g0.wait()
            pltpu.sync_copy(rows0, out_hbm.at[pl.ds(off0, block_rows)])
            g1.wait()
            pltpu.sync_copy(rows1, out_hbm.at[pl.ds(off1, block_rows)])
            return 0

        lax.fori_loop(0, nblk // 2, body, 0)

    return k(table, idx)


def _sc_scatter_add(rows, idx, n_chunks, block_rows):
    b, d = rows.shape
    dc = d // n_chunks
    bpw = b // NW
    nblk = bpw // block_rows
    rpt = N_PAD // 16
    zeros_pad = jnp.zeros((N_PAD, dc), jnp.float32)

    @functools.partial(
        pl.kernel,
        mesh=plsc.VectorSubcoreMesh(**_MESH),
        out_type=jax.ShapeDtypeStruct((2, N_PAD, d), jnp.float32),
        scratch_types=[
            pltpu.VMEM((block_rows,), jnp.int32),
            pltpu.VMEM((block_rows,), jnp.int32),
            pltpu.VMEM((block_rows, dc), jnp.float32),
            pltpu.VMEM((block_rows, dc), jnp.float32),
            pltpu.SemaphoreType.DMA,
            pltpu.SemaphoreType.DMA,
            pltpu.VMEM_SHARED((N_PAD, dc), jnp.float32),
        ],
    )
    def k(rows_hbm, idx_hbm, zero_hbm, out_hbm, idx0, idx1, rows0, rows1,
          sem0, sem1, acc):
        cid = lax.axis_index("c")
        sid = lax.axis_index("s")
        wid = sid * 2 + cid
        base = wid * bpw
        r0 = sid * rpt

        for ch in range(n_chunks):
            c0 = ch * dc
            pltpu.sync_copy(zero_hbm.at[pl.ds(r0, rpt)], acc.at[pl.ds(r0, rpt)])
            plsc.subcore_barrier()

            def body(j, _):
                off0 = base + (2 * j) * block_rows
                off1 = off0 + block_rows
                pltpu.sync_copy(idx_hbm.at[pl.ds(off0, block_rows)], idx0)
                pltpu.sync_copy(idx_hbm.at[pl.ds(off1, block_rows)], idx1)
                if n_chunks == 1:
                    g0 = pltpu.async_copy(
                        rows_hbm.at[pl.ds(off0, block_rows)], rows0, sem0)
                    g1 = pltpu.async_copy(
                        rows_hbm.at[pl.ds(off1, block_rows)], rows1, sem1)
                else:
                    g0 = pltpu.async_copy(
                        rows_hbm.at[pl.ds(off0, block_rows), pl.ds(c0, dc)],
                        rows0, sem0)
                    g1 = pltpu.async_copy(
                        rows_hbm.at[pl.ds(off1, block_rows), pl.ds(c0, dc)],
                        rows1, sem1)
                g0.wait()
                pltpu.sync_copy(rows0, acc.at[idx0], add=True)
                g1.wait()
                pltpu.sync_copy(rows1, acc.at[idx1], add=True)
                return 0

            lax.fori_loop(0, nblk // 2, body, 0)
            plsc.subcore_barrier()
            if n_chunks == 1:
                pltpu.sync_copy(acc.at[pl.ds(r0, rpt)],
                                out_hbm.at[cid, pl.ds(r0, rpt)])
            else:
                pltpu.sync_copy(
                    acc.at[pl.ds(r0, rpt)],
                    out_hbm.at[cid, pl.ds(r0, rpt), pl.ds(c0, dc)])

    return k(rows, idx, zeros_pad)


def _proj_body(x_ref, wl_ref, wr_ref, bl_ref, br_ref, xl_ref, xr_ref):
    xv = x_ref[...]
    xl_ref[...] = jnp.dot(xv, wl_ref[...], preferred_element_type=jnp.float32) + bl_ref[...]
    xr_ref[...] = jnp.dot(xv, wr_ref[...], preferred_element_type=jnp.float32) + br_ref[...]


def _proj(xin, wlT, wrT, bl, br, bn):
    n, kdim = xin.shape
    d = wlT.shape[1]
    out = jax.ShapeDtypeStruct((n, d), jnp.float32)
    return pl.pallas_call(
        _proj_body,
        grid=(n // bn,),
        in_specs=[
            pl.BlockSpec((bn, kdim), lambda i: (i, 0)),
            pl.BlockSpec((kdim, d), lambda i: (0, 0)),
            pl.BlockSpec((kdim, d), lambda i: (0, 0)),
            pl.BlockSpec((1, d), lambda i: (0, 0)),
            pl.BlockSpec((1, d), lambda i: (0, 0)),
        ],
        out_specs=[
            pl.BlockSpec((bn, d), lambda i: (i, 0)),
            pl.BlockSpec((bn, d), lambda i: (i, 0)),
        ],
        out_shape=[out, out],
    )(xin, wlT, wrT, bl.reshape(1, d), br.reshape(1, d))


def _gat_proj_body(p_ref, bias_ref, wl_ref, wr_ref, bl_ref, br_ref, xl_ref, xr_ref):
    h = jnp.tanh(p_ref[0] + p_ref[1] + bias_ref[...])
    xl_ref[...] = jnp.dot(h, wl_ref[...], preferred_element_type=jnp.float32) + bl_ref[...]
    xr_ref[...] = jnp.dot(h, wr_ref[...], preferred_element_type=jnp.float32) + br_ref[...]


def _gat_proj(parts, bias, wlT, wrT, bl, br, bn):
    _, n, kdim = parts.shape
    d = wlT.shape[1]
    out = jax.ShapeDtypeStruct((n, d), jnp.float32)
    return pl.pallas_call(
        _gat_proj_body,
        grid=(n // bn,),
        in_specs=[
            pl.BlockSpec((2, bn, kdim), lambda i: (0, i, 0)),
            pl.BlockSpec((1, kdim), lambda i: (0, 0)),
            pl.BlockSpec((kdim, d), lambda i: (0, 0)),
            pl.BlockSpec((kdim, d), lambda i: (0, 0)),
            pl.BlockSpec((1, d), lambda i: (0, 0)),
            pl.BlockSpec((1, d), lambda i: (0, 0)),
        ],
        out_specs=[
            pl.BlockSpec((bn, d), lambda i: (i, 0)),
            pl.BlockSpec((bn, d), lambda i: (i, 0)),
        ],
        out_shape=[out, out],
    )(parts, bias.reshape(1, kdim), wlT, wrT, bl.reshape(1, d), br.reshape(1, d))


def _alpha_body(xlg_ref, xrg_ref, ea_ref, we_ref, att_ref, sel_ref, o_ref):
    u = xlg_ref[...] + xrg_ref[...] + jnp.dot(
        ea_ref[...], we_ref[...], preferred_element_type=jnp.float32)
    m = jnp.maximum(u, 0.2 * u)
    o_ref[...] = jnp.dot(m * att_ref[...], sel_ref[...],
                         preferred_element_type=jnp.float32)


def _alpha(xlg, xrg, ea, weT, att_row, sel, be):
    b, d = xlg.shape
    return pl.pallas_call(
        _alpha_body,
        grid=(b // be,),
        in_specs=[
            pl.BlockSpec((be, d), lambda i: (i, 0)),
            pl.BlockSpec((be, d), lambda i: (i, 0)),
            pl.BlockSpec((be, 16), lambda i: (i, 0)),
            pl.BlockSpec((16, d), lambda i: (0, 0)),
            pl.BlockSpec((1, d), lambda i: (0, 0)),
            pl.BlockSpec((d, 128), lambda i: (0, 0)),
        ],
        out_specs=pl.BlockSpec((be, 128), lambda i: (i, 0)),
        out_shape=jax.ShapeDtypeStruct((b, 128), jnp.float32),
    )(xlg, xrg, ea, weT, att_row, sel)


def _exp_body(a_ref, g_ref, o_ref, *, be, n_valid):
    gid = pl.program_id(0) * be + lax.broadcasted_iota(jnp.int32, (be, 128), 0)
    lane = lax.broadcasted_iota(jnp.int32, (be, 128), 1)
    ok = (gid < n_valid) & (lane < 8)
    o_ref[...] = jnp.where(ok, jnp.exp(a_ref[...] - g_ref[0, 0]), 0.0)


def _exp_mask(alpha, gmax, be):
    b = alpha.shape[0]
    return pl.pallas_call(
        functools.partial(_exp_body, be=be, n_valid=EP),
        grid=(b // be,),
        in_specs=[
            pl.BlockSpec((be, 128), lambda i: (i, 0)),
            pl.BlockSpec((1, 1), lambda i: (0, 0)),
        ],
        out_specs=pl.BlockSpec((be, 128), lambda i: (i, 0)),
        out_shape=jax.ShapeDtypeStruct((b, 128), jnp.float32),
    )(alpha, gmax.reshape(1, 1))


def _weight_body(xlg_ref, ex_ref, dg_ref, bsel_ref, o_ref):
    dg = dg_ref[...]
    w16 = ex_ref[...] / jnp.where(dg > 0.0, dg, 1.0)
    wfull = jnp.dot(w16, bsel_ref[...], preferred_element_type=jnp.float32)
    o_ref[...] = wfull * xlg_ref[...]


def _weight(xlg, ex, deng, bsel, be):
    b, d = xlg.shape
    return pl.pallas_call(
        _weight_body,
        grid=(b // be,),
        in_specs=[
            pl.BlockSpec((be, d), lambda i: (i, 0)),
            pl.BlockSpec((be, 128), lambda i: (i, 0)),
            pl.BlockSpec((be, 128), lambda i: (i, 0)),
            pl.BlockSpec((128, d), lambda i: (0, 0)),
        ],
        out_specs=pl.BlockSpec((be, d), lambda i: (i, 0)),
        out_shape=jax.ShapeDtypeStruct((b, d), jnp.float32),
    )(xlg, ex, deng, bsel)


def _head_body(p_ref, bias2_ref, wih_ref, b_ref, wfc_ref, bfc_ref, o_ref):
    s = p_ref[0] + p_ref[1]
    mean = s[:, 0:128]
    for h in range(1, 8):
        mean = mean + s[:, h * 128:(h + 1) * 128]
    h2 = jnp.tanh(mean * 0.125 + bias2_ref[...])
    gates = jnp.dot(h2, wih_ref[...], preferred_element_type=jnp.float32) + b_ref[...]
    i, f, g, o = jnp.split(gates, 4, axis=1)
    c = jax.nn.sigmoid(i) * jnp.tanh(g)
    hd = jax.nn.sigmoid(o) * jnp.tanh(c)
    o_ref[...] = jnp.sum(hd * wfc_ref[...], axis=1, keepdims=True) + bfc_ref[0, 0]


def _head(parts, bias2, WihT, b, Wfc, bfc, bn):
    _, n, d = parts.shape
    return pl.pallas_call(
        _head_body,
        grid=(n // bn,),
        in_specs=[
            pl.BlockSpec((2, bn, d), lambda i: (0, i, 0)),
            pl.BlockSpec((1, 128), lambda i: (0, 0)),
            pl.BlockSpec((128, 128), lambda i: (0, 0)),
            pl.BlockSpec((1, 128), lambda i: (0, 0)),
            pl.BlockSpec((1, 32), lambda i: (0, 0)),
            pl.BlockSpec((1, 1), lambda i: (0, 0)),
        ],
        out_specs=pl.BlockSpec((bn, 1), lambda i: (i, 0)),
        out_shape=jax.ShapeDtypeStruct((n, 1), jnp.float32),
    )(parts, bias2.reshape(1, 128), WihT, b.reshape(1, 128), Wfc, bfc.reshape(1, 1))


def _selectors(d, c_per_head):
    ch = jnp.arange(d, dtype=jnp.int32) // c_per_head
    hh = jnp.arange(128, dtype=jnp.int32)
    sel = ((ch[:, None] == hh[None, :]) & (hh[None, :] < 8)).astype(jnp.float32)
    return sel, sel.T


def _pad_rows(a, n):
    return jnp.pad(a, ((0, n - a.shape[0]),) + ((0, 0),) * (a.ndim - 1))


def kernel(x, edge_index, edge_attr, Wl1, bl1, Wr1, br1, We1, att1, bias1,
           Wl2, bl2, Wr2, br2, We2, att2, bias2, Wih, Whh, bih, bhh, Wfc, bfc):
    n = x.shape[0]
    src0 = edge_index[0].astype(jnp.int32)
    dst0 = edge_index[1].astype(jnp.int32)
    ar = jnp.arange(n, dtype=jnp.int32)
    src_p = _pad_rows(jnp.concatenate([src0, ar])[:, None], EP_PAD)[:, 0]
    dst_p = _pad_rows(jnp.concatenate([dst0, ar])[:, None], EP_PAD)[:, 0]

    ea_ext = jnp.concatenate(
        [edge_attr, jnp.ones((N_EDGES, 1), jnp.float32),
         jnp.zeros((N_EDGES, 111), jnp.float32)], axis=1)
    s0 = _sc_scatter_add(_pad_rows(ea_ext, E0_PAD),
                         _pad_rows(dst0[:, None], E0_PAD)[:, 0],
                         n_chunks=1, block_rows=128)
    ssum = s0[0] + s0[1]
    cnt = ssum[:n, 16:17]
    mean_attr = ssum[:n, :16] / jnp.maximum(cnt, 1.0)
    ea_full = _pad_rows(jnp.concatenate([edge_attr, mean_attr], axis=0), EP_PAD)

    x_p = _pad_rows(x, N_PAD)

    sel1, bsel1 = _selectors(512, 64)
    xl1, xr1 = _proj(x_p, Wl1.T, Wr1.T, bl1, br1, bn=632)
    xlg1 = _sc_gather(xl1, src_p, block_rows=64)
    xrg1 = _sc_gather(xr1, dst_p, block_rows=64)
    alpha1 = _alpha(xlg1, xrg1, ea_full, We1.T, att1.reshape(1, 512), sel1, be=2048)
    gmax1 = jnp.max(alpha1[:, :8])
    ex1 = _exp_mask(alpha1, gmax1, be=2048)
    den1p = _sc_scatter_add(ex1, dst_p, n_chunks=1, block_rows=128)
    deng1 = _sc_gather(den1p[0] + den1p[1], dst_p, block_rows=128)
    w1 = _weight(xlg1, ex1, deng1, bsel1, be=2048)
    out1p = _sc_scatter_add(w1, dst_p, n_chunks=4, block_rows=64)

    sel2, bsel2 = _selectors(1024, 128)
    xl2, xr2 = _gat_proj(out1p, bias1, Wl2.T, Wr2.T, bl2, br2, bn=632)
    xlg2 = _sc_gather(xl2, src_p, block_rows=32)
    xrg2 = _sc_gather(xr2, dst_p, block_rows=32)
    alpha2 = _alpha(xlg2, xrg2, ea_full, We2.T, att2.reshape(1, 1024), sel2, be=1024)
    gmax2 = jnp.max(alpha2[:, :8])
    ex2 = _exp_mask(alpha2, gmax2, be=2048)
    den2p = _sc_scatter_add(ex2, dst_p, n_chunks=1, block_rows=128)
    deng2 = _sc_gather(den2p[0] + den2p[1], dst_p, block_rows=128)
    w2 = _weight(xlg2, ex2, deng2, bsel2, be=1024)
    out2p = _sc_scatter_add(w2, dst_p, n_chunks=8, block_rows=64)

    y = _head(out2p, bias2, Wih.T, bih + bhh, Wfc, bfc, bn=632)
    return y[:n]


# speedup vs baseline: 8.7570x; 1.1908x over previous
import functools

import jax
import jax.numpy as jnp
from jax import lax
from jax.experimental import pallas as pl
from jax.experimental.pallas import tpu as pltpu
from jax.experimental.pallas import tpu_sc as plsc

N_NODES = 10000
N_PAD = 10112
N_EDGES = 160000
E0_PAD = 163840
EP = 170000
EP_PAD = 172032
NW = 32
_MESH = dict(core_axis_name="c", subcore_axis_name="s")


def _sc_gather(table, idx, block_rows):
    n_rows, d = table.shape
    b = idx.shape[0]
    bpw = b // NW
    nblk = bpw // block_rows

    assert nblk % 2 == 0

    @functools.partial(
        pl.kernel,
        mesh=plsc.VectorSubcoreMesh(**_MESH),
        out_type=jax.ShapeDtypeStruct((b, d), jnp.float32),
        scratch_types=[
            pltpu.VMEM((block_rows,), jnp.int32),
            pltpu.VMEM((block_rows,), jnp.int32),
            pltpu.VMEM((block_rows, d), jnp.float32),
            pltpu.VMEM((block_rows, d), jnp.float32),
            pltpu.SemaphoreType.DMA,
            pltpu.SemaphoreType.DMA,
        ],
    )
    def k(table_hbm, idx_hbm, out_hbm, idx0, idx1, rows0, rows1, sem0, sem1):
        wid = lax.axis_index("s") * 2 + lax.axis_index("c")
        base = wid * bpw

        def body(j, _):
            off0 = base + (2 * j) * block_rows
            off1 = off0 + block_rows
            pltpu.sync_copy(idx_hbm.at[pl.ds(off0, block_rows)], idx0)
            pltpu.sync_copy(idx_hbm.at[pl.ds(off1, block_rows)], idx1)
            g0 = pltpu.async_copy(table_hbm.at[idx0], rows0, sem0)
            g1 = pltpu.async_copy(table_hbm.at[idx1], rows1, sem1)
            g0.wait()
            pltpu.sync_copy(rows0, out_hbm.at[pl.ds(off0, block_rows)])
            g1.wait()
            pltpu.sync_copy(rows1, out_hbm.at[pl.ds(off1, block_rows)])
            return 0

        lax.fori_loop(0, nblk // 2, body, 0)

    return k(table, idx)


def _sc_scatter_add(rows, idx, n_chunks, block_rows):
    b, d = rows.shape
    dc = d // n_chunks
    bpw = b // NW
    nblk = bpw // block_rows
    rpt = N_PAD // 16
    zeros_pad = jnp.zeros((N_PAD, dc), jnp.float32)

    @functools.partial(
        pl.kernel,
        mesh=plsc.VectorSubcoreMesh(**_MESH),
        out_type=jax.ShapeDtypeStruct((2, N_PAD, d), jnp.float32),
        scratch_types=[
            pltpu.VMEM((block_rows,), jnp.int32),
            pltpu.VMEM((block_rows,), jnp.int32),
            pltpu.VMEM((block_rows, dc), jnp.float32),
            pltpu.VMEM((block_rows, dc), jnp.float32),
            pltpu.SemaphoreType.DMA,
            pltpu.SemaphoreType.DMA,
            pltpu.VMEM_SHARED((N_PAD, dc), jnp.float32),
        ],
    )
    def k(rows_hbm, idx_hbm, zero_hbm, out_hbm, idx0, idx1, rows0, rows1,
          sem0, sem1, acc):
        cid = lax.axis_index("c")
        sid = lax.axis_index("s")
        wid = sid * 2 + cid
        base = wid * bpw
        r0 = sid * rpt

        for ch in range(n_chunks):
            c0 = ch * dc
            pltpu.sync_copy(zero_hbm.at[pl.ds(r0, rpt)], acc.at[pl.ds(r0, rpt)])
            plsc.subcore_barrier()

            def body(j, _):
                off0 = base + (2 * j) * block_rows
                off1 = off0 + block_rows
                pltpu.sync_copy(idx_hbm.at[pl.ds(off0, block_rows)], idx0)
                pltpu.sync_copy(idx_hbm.at[pl.ds(off1, block_rows)], idx1)
                if n_chunks == 1:
                    g0 = pltpu.async_copy(
                        rows_hbm.at[pl.ds(off0, block_rows)], rows0, sem0)
                    g1 = pltpu.async_copy(
                        rows_hbm.at[pl.ds(off1, block_rows)], rows1, sem1)
                else:
                    g0 = pltpu.async_copy(
                        rows_hbm.at[pl.ds(off0, block_rows), pl.ds(c0, dc)],
                        rows0, sem0)
                    g1 = pltpu.async_copy(
                        rows_hbm.at[pl.ds(off1, block_rows), pl.ds(c0, dc)],
                        rows1, sem1)
                g0.wait()
                pltpu.sync_copy(rows0, acc.at[idx0], add=True)
                g1.wait()
                pltpu.sync_copy(rows1, acc.at[idx1], add=True)
                return 0

            lax.fori_loop(0, nblk // 2, body, 0)
            plsc.subcore_barrier()
            if n_chunks == 1:
                pltpu.sync_copy(acc.at[pl.ds(r0, rpt)],
                                out_hbm.at[cid, pl.ds(r0, rpt)])
            else:
                pltpu.sync_copy(
                    acc.at[pl.ds(r0, rpt)],
                    out_hbm.at[cid, pl.ds(r0, rpt), pl.ds(c0, dc)])

    return k(rows, idx, zeros_pad)


def _proj_body(x_ref, wl_ref, wr_ref, bl_ref, br_ref, xl_ref, xr_ref):
    xv = x_ref[...]
    xl_ref[...] = jnp.dot(xv, wl_ref[...], preferred_element_type=jnp.float32) + bl_ref[...]
    xr_ref[...] = jnp.dot(xv, wr_ref[...], preferred_element_type=jnp.float32) + br_ref[...]


def _proj(xin, wlT, wrT, bl, br, bn):
    n, kdim = xin.shape
    d = wlT.shape[1]
    out = jax.ShapeDtypeStruct((n, d), jnp.float32)
    return pl.pallas_call(
        _proj_body,
        grid=(n // bn,),
        in_specs=[
            pl.BlockSpec((bn, kdim), lambda i: (i, 0)),
            pl.BlockSpec((kdim, d), lambda i: (0, 0)),
            pl.BlockSpec((kdim, d), lambda i: (0, 0)),
            pl.BlockSpec((1, d), lambda i: (0, 0)),
            pl.BlockSpec((1, d), lambda i: (0, 0)),
        ],
        out_specs=[
            pl.BlockSpec((bn, d), lambda i: (i, 0)),
            pl.BlockSpec((bn, d), lambda i: (i, 0)),
        ],
        out_shape=[out, out],
    )(xin, wlT, wrT, bl.reshape(1, d), br.reshape(1, d))


def _gat_proj_body(p_ref, dp_ref, bsel_ref, bias_ref, wl_ref, wr_ref, bl_ref,
                   br_ref, xl_ref, xr_ref):
    den = jnp.dot(dp_ref[0] + dp_ref[1], bsel_ref[...],
                  preferred_element_type=jnp.float32)
    den = jnp.maximum(den, 1e-30)
    h = jnp.tanh((p_ref[0] + p_ref[1]) / den + bias_ref[...])
    xl_ref[...] = jnp.dot(h, wl_ref[...], preferred_element_type=jnp.float32) + bl_ref[...]
    xr_ref[...] = jnp.dot(h, wr_ref[...], preferred_element_type=jnp.float32) + br_ref[...]


def _gat_proj(parts, denp, bsel, bias, wlT, wrT, bl, br, bn):
    _, n, kdim = parts.shape
    d = wlT.shape[1]
    out = jax.ShapeDtypeStruct((n, d), jnp.float32)
    return pl.pallas_call(
        _gat_proj_body,
        grid=(n // bn,),
        in_specs=[
            pl.BlockSpec((2, bn, kdim), lambda i: (0, i, 0)),
            pl.BlockSpec((2, bn, 128), lambda i: (0, i, 0)),
            pl.BlockSpec((128, kdim), lambda i: (0, 0)),
            pl.BlockSpec((1, kdim), lambda i: (0, 0)),
            pl.BlockSpec((kdim, d), lambda i: (0, 0)),
            pl.BlockSpec((kdim, d), lambda i: (0, 0)),
            pl.BlockSpec((1, d), lambda i: (0, 0)),
            pl.BlockSpec((1, d), lambda i: (0, 0)),
        ],
        out_specs=[
            pl.BlockSpec((bn, d), lambda i: (i, 0)),
            pl.BlockSpec((bn, d), lambda i: (i, 0)),
        ],
        out_shape=[out, out],
    )(parts, denp, bsel, bias.reshape(1, kdim), wlT, wrT,
      bl.reshape(1, d), br.reshape(1, d))


def _alpha_body(xlg_ref, xrg_ref, ea_ref, we_ref, att_ref, sel_ref, o_ref):
    u = xlg_ref[...] + xrg_ref[...] + jnp.dot(
        ea_ref[...], we_ref[...], preferred_element_type=jnp.float32)
    m = jnp.maximum(u, 0.2 * u)
    o_ref[...] = jnp.dot(m * att_ref[...], sel_ref[...],
                         preferred_element_type=jnp.float32)


def _alpha(xlg, xrg, ea, weT, att_row, sel, be):
    b, d = xlg.shape
    return pl.pallas_call(
        _alpha_body,
        grid=(b // be,),
        in_specs=[
            pl.BlockSpec((be, d), lambda i: (i, 0)),
            pl.BlockSpec((be, d), lambda i: (i, 0)),
            pl.BlockSpec((be, 16), lambda i: (i, 0)),
            pl.BlockSpec((16, d), lambda i: (0, 0)),
            pl.BlockSpec((1, d), lambda i: (0, 0)),
            pl.BlockSpec((d, 128), lambda i: (0, 0)),
        ],
        out_specs=pl.BlockSpec((be, 128), lambda i: (i, 0)),
        out_shape=jax.ShapeDtypeStruct((b, 128), jnp.float32),
    )(xlg, xrg, ea, weT, att_row, sel)


def _ew_body(xlg_ref, a_ref, g_ref, bsel_ref, w_ref, ex_ref, *, be, n_valid):
    gid = pl.program_id(0) * be + lax.broadcasted_iota(jnp.int32, (be, 128), 0)
    lane = lax.broadcasted_iota(jnp.int32, (be, 128), 1)
    ok = (gid < n_valid) & (lane < 8)
    ex = jnp.where(ok, jnp.exp(a_ref[...] - g_ref[0, 0]), 0.0)
    ex_ref[...] = ex
    w_ref[...] = jnp.dot(ex, bsel_ref[...],
                         preferred_element_type=jnp.float32) * xlg_ref[...]


def _exp_weight(xlg, alpha, gmax, bsel, be):
    b, d = xlg.shape
    return pl.pallas_call(
        functools.partial(_ew_body, be=be, n_valid=EP),
        grid=(b // be,),
        in_specs=[
            pl.BlockSpec((be, d), lambda i: (i, 0)),
            pl.BlockSpec((be, 128), lambda i: (i, 0)),
            pl.BlockSpec((1, 1), lambda i: (0, 0)),
            pl.BlockSpec((128, d), lambda i: (0, 0)),
        ],
        out_specs=[
            pl.BlockSpec((be, d), lambda i: (i, 0)),
            pl.BlockSpec((be, 128), lambda i: (i, 0)),
        ],
        out_shape=[jax.ShapeDtypeStruct((b, d), jnp.float32),
                   jax.ShapeDtypeStruct((b, 128), jnp.float32)],
    )(xlg, alpha, gmax.reshape(1, 1), bsel)


def _head_body(p_ref, dp_ref, bsel_ref, bias2_ref, wih_ref, b_ref, wfc_ref,
               bfc_ref, o_ref):
    den = jnp.dot(dp_ref[0] + dp_ref[1], bsel_ref[...],
                  preferred_element_type=jnp.float32)
    s = (p_ref[0] + p_ref[1]) / jnp.maximum(den, 1e-30)
    mean = s[:, 0:128]
    for h in range(1, 8):
        mean = mean + s[:, h * 128:(h + 1) * 128]
    h2 = jnp.tanh(mean * 0.125 + bias2_ref[...])
    gates = jnp.dot(h2, wih_ref[...], preferred_element_type=jnp.float32) + b_ref[...]
    i, f, g, o = jnp.split(gates, 4, axis=1)
    c = jax.nn.sigmoid(i) * jnp.tanh(g)
    hd = jax.nn.sigmoid(o) * jnp.tanh(c)
    o_ref[...] = jnp.sum(hd * wfc_ref[...], axis=1, keepdims=True) + bfc_ref[0, 0]


def _head(parts, denp, bsel, bias2, WihT, b, Wfc, bfc, bn):
    _, n, d = parts.shape
    return pl.pallas_call(
        _head_body,
        grid=(n // bn,),
        in_specs=[
            pl.BlockSpec((2, bn, d), lambda i: (0, i, 0)),
            pl.BlockSpec((2, bn, 128), lambda i: (0, i, 0)),
            pl.BlockSpec((128, d), lambda i: (0, 0)),
            pl.BlockSpec((1, 128), lambda i: (0, 0)),
            pl.BlockSpec((128, 128), lambda i: (0, 0)),
            pl.BlockSpec((1, 128), lambda i: (0, 0)),
            pl.BlockSpec((1, 32), lambda i: (0, 0)),
            pl.BlockSpec((1, 1), lambda i: (0, 0)),
        ],
        out_specs=pl.BlockSpec((bn, 1), lambda i: (i, 0)),
        out_shape=jax.ShapeDtypeStruct((n, 1), jnp.float32),
    )(parts, denp, bsel, bias2.reshape(1, 128), WihT, b.reshape(1, 128),
      Wfc, bfc.reshape(1, 1))


def _selectors(d, c_per_head):
    ch = jnp.arange(d, dtype=jnp.int32) // c_per_head
    hh = jnp.arange(128, dtype=jnp.int32)
    sel = ((ch[:, None] == hh[None, :]) & (hh[None, :] < 8)).astype(jnp.float32)
    return sel, sel.T


def _pad_rows(a, n):
    return jnp.pad(a, ((0, n - a.shape[0]),) + ((0, 0),) * (a.ndim - 1))


def kernel(x, edge_index, edge_attr, Wl1, bl1, Wr1, br1, We1, att1, bias1,
           Wl2, bl2, Wr2, br2, We2, att2, bias2, Wih, Whh, bih, bhh, Wfc, bfc):
    n = x.shape[0]
    src0 = edge_index[0].astype(jnp.int32)
    dst0 = edge_index[1].astype(jnp.int32)
    ar = jnp.arange(n, dtype=jnp.int32)
    src_p = _pad_rows(jnp.concatenate([src0, ar])[:, None], EP_PAD)[:, 0]
    dst_p = _pad_rows(jnp.concatenate([dst0, ar])[:, None], EP_PAD)[:, 0]

    ea_ext = jnp.concatenate(
        [edge_attr, jnp.ones((N_EDGES, 1), jnp.float32),
         jnp.zeros((N_EDGES, 111), jnp.float32)], axis=1)
    s0 = _sc_scatter_add(_pad_rows(ea_ext, E0_PAD),
                         _pad_rows(dst0[:, None], E0_PAD)[:, 0],
                         n_chunks=1, block_rows=128)
    ssum = s0[0] + s0[1]
    cnt = ssum[:n, 16:17]
    mean_attr = ssum[:n, :16] / jnp.maximum(cnt, 1.0)
    ea_full = _pad_rows(jnp.concatenate([edge_attr, mean_attr], axis=0), EP_PAD)

    x_p = _pad_rows(x, N_PAD)

    sel1, bsel1 = _selectors(512, 64)
    xl1, xr1 = _proj(x_p, Wl1.T, Wr1.T, bl1, br1, bn=632)
    xlg1 = _sc_gather(xl1, src_p, block_rows=96)
    xrg1 = _sc_gather(xr1, dst_p, block_rows=96)
    alpha1 = _alpha(xlg1, xrg1, ea_full, We1.T, att1.reshape(1, 512), sel1, be=2048)
    gmax1 = jnp.max(alpha1[:, :8])
    w1, ex1 = _exp_weight(xlg1, alpha1, gmax1, bsel1, be=2048)
    den1p = _sc_scatter_add(ex1, dst_p, n_chunks=1, block_rows=128)
    out1p = _sc_scatter_add(w1, dst_p, n_chunks=4, block_rows=128)

    sel2, bsel2 = _selectors(1024, 128)
    xl2, xr2 = _gat_proj(out1p, den1p, bsel1, bias1, Wl2.T, Wr2.T, bl2, br2,
                         bn=632)
    xlg2 = _sc_gather(xl2, src_p, block_rows=48)
    xrg2 = _sc_gather(xr2, dst_p, block_rows=48)
    alpha2 = _alpha(xlg2, xrg2, ea_full, We2.T, att2.reshape(1, 1024), sel2, be=1024)
    gmax2 = jnp.max(alpha2[:, :8])
    w2, ex2 = _exp_weight(xlg2, alpha2, gmax2, bsel2, be=1024)
    den2p = _sc_scatter_add(ex2, dst_p, n_chunks=1, block_rows=128)
    out2p = _sc_scatter_add(w2, dst_p, n_chunks=8, block_rows=128)

    y = _head(out2p, den2p, bsel2, bias2, Wih.T, bih + bhh, Wfc, bfc, bn=632)
    return y[:n]


# speedup vs baseline: 8.9927x; 1.0269x over previous
import functools

import jax
import jax.numpy as jnp
from jax import lax
from jax.experimental import pallas as pl
from jax.experimental.pallas import tpu as pltpu
from jax.experimental.pallas import tpu_sc as plsc

N_NODES = 10000
N_PAD = 10112
N_EDGES = 160000
E0_PAD = 163840
EP = 170000
EP_PAD = 172032
NW = 32
_MESH = dict(core_axis_name="c", subcore_axis_name="s")


def _sc_gather(table, idx, block_rows):
    n_rows, d = table.shape
    b = idx.shape[0]
    bpw = b // NW
    nblk = bpw // block_rows

    assert nblk % 2 == 0

    @functools.partial(
        pl.kernel,
        mesh=plsc.VectorSubcoreMesh(**_MESH),
        out_type=jax.ShapeDtypeStruct((b, d), jnp.float32),
        scratch_types=[
            pltpu.VMEM((block_rows,), jnp.int32),
            pltpu.VMEM((block_rows,), jnp.int32),
            pltpu.VMEM((block_rows, d), jnp.float32),
            pltpu.VMEM((block_rows, d), jnp.float32),
            pltpu.SemaphoreType.DMA,
            pltpu.SemaphoreType.DMA,
        ],
    )
    def k(table_hbm, idx_hbm, out_hbm, idx0, idx1, rows0, rows1, sem0, sem1):
        wid = lax.axis_index("s") * 2 + lax.axis_index("c")
        base = wid * bpw

        def body(j, _):
            off0 = base + (2 * j) * block_rows
            off1 = off0 + block_rows
            pltpu.sync_copy(idx_hbm.at[pl.ds(off0, block_rows)], idx0)
            pltpu.sync_copy(idx_hbm.at[pl.ds(off1, block_rows)], idx1)

            @pl.when(j > 0)
            def _():
                pltpu.make_async_copy(rows0, out_hbm.at[pl.ds(off0, block_rows)],
                                      sem0).wait()
                pltpu.make_async_copy(rows1, out_hbm.at[pl.ds(off1, block_rows)],
                                      sem1).wait()

            g0 = pltpu.async_copy(table_hbm.at[idx0], rows0, sem0)
            g1 = pltpu.async_copy(table_hbm.at[idx1], rows1, sem1)
            g0.wait()
            pltpu.async_copy(rows0, out_hbm.at[pl.ds(off0, block_rows)], sem0)
            g1.wait()
            pltpu.async_copy(rows1, out_hbm.at[pl.ds(off1, block_rows)], sem1)
            return 0

        lax.fori_loop(0, nblk // 2, body, 0)
        last0 = base + (nblk - 2) * block_rows
        pltpu.make_async_copy(rows0, out_hbm.at[pl.ds(last0, block_rows)],
                              sem0).wait()
        pltpu.make_async_copy(rows1, out_hbm.at[pl.ds(last0 + block_rows,
                                                      block_rows)], sem1).wait()

    return k(table, idx)


def _sc_scatter_add(rows, idx, n_chunks, block_rows):
    b, d = rows.shape
    dc = d // n_chunks
    bpw = b // NW
    nblk = bpw // block_rows
    rpt = N_PAD // 16
    zeros_pad = jnp.zeros((N_PAD, dc), jnp.float32)

    @functools.partial(
        pl.kernel,
        mesh=plsc.VectorSubcoreMesh(**_MESH),
        out_type=jax.ShapeDtypeStruct((2, N_PAD, d), jnp.float32),
        scratch_types=[
            pltpu.VMEM((block_rows,), jnp.int32),
            pltpu.VMEM((block_rows,), jnp.int32),
            pltpu.VMEM((block_rows, dc), jnp.float32),
            pltpu.VMEM((block_rows, dc), jnp.float32),
            pltpu.SemaphoreType.DMA,
            pltpu.SemaphoreType.DMA,
            pltpu.SemaphoreType.DMA,
            pltpu.SemaphoreType.DMA,
            pltpu.VMEM_SHARED((N_PAD, dc), jnp.float32),
        ],
    )
    def k(rows_hbm, idx_hbm, zero_hbm, out_hbm, idx0, idx1, rows0, rows1,
          sem0, sem1, sema0, sema1, acc):
        cid = lax.axis_index("c")
        sid = lax.axis_index("s")
        wid = sid * 2 + cid
        base = wid * bpw
        r0 = sid * rpt

        for ch in range(n_chunks):
            c0 = ch * dc
            pltpu.sync_copy(zero_hbm.at[pl.ds(r0, rpt)], acc.at[pl.ds(r0, rpt)])
            plsc.subcore_barrier()

            def body(j, _):
                off0 = base + (2 * j) * block_rows
                off1 = off0 + block_rows

                @pl.when(j > 0)
                def _():
                    pltpu.make_async_copy(rows0, acc.at[idx0], sema0).wait()
                    pltpu.make_async_copy(rows1, acc.at[idx1], sema1).wait()

                pltpu.sync_copy(idx_hbm.at[pl.ds(off0, block_rows)], idx0)
                pltpu.sync_copy(idx_hbm.at[pl.ds(off1, block_rows)], idx1)
                if n_chunks == 1:
                    g0 = pltpu.async_copy(
                        rows_hbm.at[pl.ds(off0, block_rows)], rows0, sem0)
                    g1 = pltpu.async_copy(
                        rows_hbm.at[pl.ds(off1, block_rows)], rows1, sem1)
                else:
                    g0 = pltpu.async_copy(
                        rows_hbm.at[pl.ds(off0, block_rows), pl.ds(c0, dc)],
                        rows0, sem0)
                    g1 = pltpu.async_copy(
                        rows_hbm.at[pl.ds(off1, block_rows), pl.ds(c0, dc)],
                        rows1, sem1)
                g0.wait()
                pltpu.async_copy(rows0, acc.at[idx0], sema0, add=True)
                g1.wait()
                pltpu.async_copy(rows1, acc.at[idx1], sema1, add=True)
                return 0

            lax.fori_loop(0, nblk // 2, body, 0)
            pltpu.make_async_copy(rows0, acc.at[idx0], sema0).wait()
            pltpu.make_async_copy(rows1, acc.at[idx1], sema1).wait()
            plsc.subcore_barrier()
            if n_chunks == 1:
                pltpu.sync_copy(acc.at[pl.ds(r0, rpt)],
                                out_hbm.at[cid, pl.ds(r0, rpt)])
            else:
                pltpu.sync_copy(
                    acc.at[pl.ds(r0, rpt)],
                    out_hbm.at[cid, pl.ds(r0, rpt), pl.ds(c0, dc)])

    return k(rows, idx, zeros_pad)


def _proj_body(x_ref, wl_ref, wr_ref, bl_ref, br_ref, xl_ref, xr_ref):
    xv = x_ref[...]
    xl_ref[...] = jnp.dot(xv, wl_ref[...], preferred_element_type=jnp.float32) + bl_ref[...]
    xr_ref[...] = jnp.dot(xv, wr_ref[...], preferred_element_type=jnp.float32) + br_ref[...]


def _proj(xin, wlT, wrT, bl, br, bn):
    n, kdim = xin.shape
    d = wlT.shape[1]
    out = jax.ShapeDtypeStruct((n, d), jnp.float32)
    return pl.pallas_call(
        _proj_body,
        grid=(n // bn,),
        in_specs=[
            pl.BlockSpec((bn, kdim), lambda i: (i, 0)),
            pl.BlockSpec((kdim, d), lambda i: (0, 0)),
            pl.BlockSpec((kdim, d), lambda i: (0, 0)),
            pl.BlockSpec((1, d), lambda i: (0, 0)),
            pl.BlockSpec((1, d), lambda i: (0, 0)),
        ],
        out_specs=[
            pl.BlockSpec((bn, d), lambda i: (i, 0)),
            pl.BlockSpec((bn, d), lambda i: (i, 0)),
        ],
        out_shape=[out, out],
    )(xin, wlT, wrT, bl.reshape(1, d), br.reshape(1, d))


def _gat_proj_body(p_ref, dp_ref, bsel_ref, bias_ref, wl_ref, wr_ref, bl_ref,
                   br_ref, xl_ref, xr_ref):
    den = jnp.dot(dp_ref[0] + dp_ref[1], bsel_ref[...],
                  preferred_element_type=jnp.float32)
    den = jnp.maximum(den, 1e-30)
    h = jnp.tanh((p_ref[0] + p_ref[1]) / den + bias_ref[...])
    xl_ref[...] = jnp.dot(h, wl_ref[...], preferred_element_type=jnp.float32) + bl_ref[...]
    xr_ref[...] = jnp.dot(h, wr_ref[...], preferred_element_type=jnp.float32) + br_ref[...]


def _gat_proj(parts, denp, bsel, bias, wlT, wrT, bl, br, bn):
    _, n, kdim = parts.shape
    d = wlT.shape[1]
    out = jax.ShapeDtypeStruct((n, d), jnp.float32)
    return pl.pallas_call(
        _gat_proj_body,
        grid=(n // bn,),
        in_specs=[
            pl.BlockSpec((2, bn, kdim), lambda i: (0, i, 0)),
            pl.BlockSpec((2, bn, 128), lambda i: (0, i, 0)),
            pl.BlockSpec((128, kdim), lambda i: (0, 0)),
            pl.BlockSpec((1, kdim), lambda i: (0, 0)),
            pl.BlockSpec((kdim, d), lambda i: (0, 0)),
            pl.BlockSpec((kdim, d), lambda i: (0, 0)),
            pl.BlockSpec((1, d), lambda i: (0, 0)),
            pl.BlockSpec((1, d), lambda i: (0, 0)),
        ],
        out_specs=[
            pl.BlockSpec((bn, d), lambda i: (i, 0)),
            pl.BlockSpec((bn, d), lambda i: (i, 0)),
        ],
        out_shape=[out, out],
    )(parts, denp, bsel, bias.reshape(1, kdim), wlT, wrT,
      bl.reshape(1, d), br.reshape(1, d))


def _alpha_body(xlg_ref, xrg_ref, ea_ref, we_ref, att_ref, sel_ref, o_ref):
    u = xlg_ref[...] + xrg_ref[...] + jnp.dot(
        ea_ref[...], we_ref[...], preferred_element_type=jnp.float32)
    m = jnp.maximum(u, 0.2 * u)
    o_ref[...] = jnp.dot(m * att_ref[...], sel_ref[...],
                         preferred_element_type=jnp.float32)


def _alpha(xlg, xrg, ea, weT, att_row, sel, be):
    b, d = xlg.shape
    return pl.pallas_call(
        _alpha_body,
        grid=(b // be,),
        in_specs=[
            pl.BlockSpec((be, d), lambda i: (i, 0)),
            pl.BlockSpec((be, d), lambda i: (i, 0)),
            pl.BlockSpec((be, 16), lambda i: (i, 0)),
            pl.BlockSpec((16, d), lambda i: (0, 0)),
            pl.BlockSpec((1, d), lambda i: (0, 0)),
            pl.BlockSpec((d, 128), lambda i: (0, 0)),
        ],
        out_specs=pl.BlockSpec((be, 128), lambda i: (i, 0)),
        out_shape=jax.ShapeDtypeStruct((b, 128), jnp.float32),
    )(xlg, xrg, ea, weT, att_row, sel)


def _ew_body(xlg_ref, a_ref, g_ref, bsel_ref, w_ref, ex_ref, *, be, n_valid):
    gid = pl.program_id(0) * be + lax.broadcasted_iota(jnp.int32, (be, 128), 0)
    lane = lax.broadcasted_iota(jnp.int32, (be, 128), 1)
    ok = (gid < n_valid) & (lane < 8)
    ex = jnp.where(ok, jnp.exp(a_ref[...] - g_ref[0, 0]), 0.0)
    ex_ref[...] = ex
    w_ref[...] = jnp.dot(ex, bsel_ref[...],
                         preferred_element_type=jnp.float32) * xlg_ref[...]


def _exp_weight(xlg, alpha, gmax, bsel, be):
    b, d = xlg.shape
    return pl.pallas_call(
        functools.partial(_ew_body, be=be, n_valid=EP),
        grid=(b // be,),
        in_specs=[
            pl.BlockSpec((be, d), lambda i: (i, 0)),
            pl.BlockSpec((be, 128), lambda i: (i, 0)),
            pl.BlockSpec((1, 1), lambda i: (0, 0)),
            pl.BlockSpec((128, d), lambda i: (0, 0)),
        ],
        out_specs=[
            pl.BlockSpec((be, d), lambda i: (i, 0)),
            pl.BlockSpec((be, 128), lambda i: (i, 0)),
        ],
        out_shape=[jax.ShapeDtypeStruct((b, d), jnp.float32),
                   jax.ShapeDtypeStruct((b, 128), jnp.float32)],
    )(xlg, alpha, gmax.reshape(1, 1), bsel)


def _head_body(p_ref, dp_ref, bsel_ref, bias2_ref, wih_ref, b_ref, wfc_ref,
               bfc_ref, o_ref):
    den = jnp.dot(dp_ref[0] + dp_ref[1], bsel_ref[...],
                  preferred_element_type=jnp.float32)
    s = (p_ref[0] + p_ref[1]) / jnp.maximum(den, 1e-30)
    mean = s[:, 0:128]
    for h in range(1, 8):
        mean = mean + s[:, h * 128:(h + 1) * 128]
    h2 = jnp.tanh(mean * 0.125 + bias2_ref[...])
    gates = jnp.dot(h2, wih_ref[...], preferred_element_type=jnp.float32) + b_ref[...]
    i, f, g, o = jnp.split(gates, 4, axis=1)
    c = jax.nn.sigmoid(i) * jnp.tanh(g)
    hd = jax.nn.sigmoid(o) * jnp.tanh(c)
    o_ref[...] = jnp.sum(hd * wfc_ref[...], axis=1, keepdims=True) + bfc_ref[0, 0]


def _head(parts, denp, bsel, bias2, WihT, b, Wfc, bfc, bn):
    _, n, d = parts.shape
    return pl.pallas_call(
        _head_body,
        grid=(n // bn,),
        in_specs=[
            pl.BlockSpec((2, bn, d), lambda i: (0, i, 0)),
            pl.BlockSpec((2, bn, 128), lambda i: (0, i, 0)),
            pl.BlockSpec((128, d), lambda i: (0, 0)),
            pl.BlockSpec((1, 128), lambda i: (0, 0)),
            pl.BlockSpec((128, 128), lambda i: (0, 0)),
            pl.BlockSpec((1, 128), lambda i: (0, 0)),
            pl.BlockSpec((1, 32), lambda i: (0, 0)),
            pl.BlockSpec((1, 1), lambda i: (0, 0)),
        ],
        out_specs=pl.BlockSpec((bn, 1), lambda i: (i, 0)),
        out_shape=jax.ShapeDtypeStruct((n, 1), jnp.float32),
    )(parts, denp, bsel, bias2.reshape(1, 128), WihT, b.reshape(1, 128),
      Wfc, bfc.reshape(1, 1))


def _selectors(d, c_per_head):
    ch = jnp.arange(d, dtype=jnp.int32) // c_per_head
    hh = jnp.arange(128, dtype=jnp.int32)
    sel = ((ch[:, None] == hh[None, :]) & (hh[None, :] < 8)).astype(jnp.float32)
    return sel, sel.T


def _pad_rows(a, n):
    return jnp.pad(a, ((0, n - a.shape[0]),) + ((0, 0),) * (a.ndim - 1))


def kernel(x, edge_index, edge_attr, Wl1, bl1, Wr1, br1, We1, att1, bias1,
           Wl2, bl2, Wr2, br2, We2, att2, bias2, Wih, Whh, bih, bhh, Wfc, bfc):
    n = x.shape[0]
    src0 = edge_index[0].astype(jnp.int32)
    dst0 = edge_index[1].astype(jnp.int32)
    ar = jnp.arange(n, dtype=jnp.int32)
    src_p = _pad_rows(jnp.concatenate([src0, ar])[:, None], EP_PAD)[:, 0]
    dst_p = _pad_rows(jnp.concatenate([dst0, ar])[:, None], EP_PAD)[:, 0]

    ea_ext = jnp.concatenate(
        [edge_attr, jnp.ones((N_EDGES, 1), jnp.float32),
         jnp.zeros((N_EDGES, 111), jnp.float32)], axis=1)
    s0 = _sc_scatter_add(_pad_rows(ea_ext, E0_PAD),
                         _pad_rows(dst0[:, None], E0_PAD)[:, 0],
                         n_chunks=1, block_rows=128)
    ssum = s0[0] + s0[1]
    cnt = ssum[:n, 16:17]
    mean_attr = ssum[:n, :16] / jnp.maximum(cnt, 1.0)
    ea_full = _pad_rows(jnp.concatenate([edge_attr, mean_attr], axis=0), EP_PAD)

    x_p = _pad_rows(x, N_PAD)

    sel1, bsel1 = _selectors(512, 64)
    xl1, xr1 = _proj(x_p, Wl1.T, Wr1.T, bl1, br1, bn=632)
    xlg1 = _sc_gather(xl1, src_p, block_rows=96)
    xrg1 = _sc_gather(xr1, dst_p, block_rows=96)
    alpha1 = _alpha(xlg1, xrg1, ea_full, We1.T, att1.reshape(1, 512), sel1, be=2048)
    gmax1 = jnp.max(alpha1[:, :8])
    w1, ex1 = _exp_weight(xlg1, alpha1, gmax1, bsel1, be=2048)
    den1p = _sc_scatter_add(ex1, dst_p, n_chunks=1, block_rows=128)
    out1p = _sc_scatter_add(w1, dst_p, n_chunks=4, block_rows=128)

    sel2, bsel2 = _selectors(1024, 128)
    xl2, xr2 = _gat_proj(out1p, den1p, bsel1, bias1, Wl2.T, Wr2.T, bl2, br2,
                         bn=632)
    xlg2 = _sc_gather(xl2, src_p, block_rows=48)
    xrg2 = _sc_gather(xr2, dst_p, block_rows=48)
    alpha2 = _alpha(xlg2, xrg2, ea_full, We2.T, att2.reshape(1, 1024), sel2, be=1024)
    gmax2 = jnp.max(alpha2[:, :8])
    w2, ex2 = _exp_weight(xlg2, alpha2, gmax2, bsel2, be=1024)
    den2p = _sc_scatter_add(ex2, dst_p, n_chunks=1, block_rows=128)
    out2p = _sc_scatter_add(w2, dst_p, n_chunks=8, block_rows=128)

    y = _head(out2p, den2p, bsel2, bias2, Wih.T, bih + bhh, Wfc, bfc, bn=632)
    return y[:n]


# speedup vs baseline: 9.9579x; 1.1073x over previous
import functools

import jax
import jax.numpy as jnp
from jax import lax
from jax.experimental import pallas as pl
from jax.experimental.pallas import tpu as pltpu
from jax.experimental.pallas import tpu_sc as plsc

N_NODES = 10000
N_PAD = 10112
N_EDGES = 160000
E0_PAD = 163840
EP = 170000
EP_PAD = 172032
NW = 32
_MESH = dict(core_axis_name="c", subcore_axis_name="s")


def _sc_gather(table, idx, block_rows):
    n_rows, d = table.shape
    b = idx.shape[0]
    bpw = b // NW
    nblk = bpw // block_rows
    assert nblk % 4 == 0 and bpw == nblk * block_rows
    idx3 = idx.reshape(NW, nblk, block_rows)

    @functools.partial(
        pl.kernel,
        mesh=plsc.VectorSubcoreMesh(**_MESH),
        out_type=jax.ShapeDtypeStruct((b, d), jnp.float32),
        scratch_types=[
            pltpu.VMEM((nblk, block_rows), jnp.int32),
            pltpu.VMEM((block_rows, d), jnp.float32),
            pltpu.VMEM((block_rows, d), jnp.float32),
            pltpu.VMEM((block_rows, d), jnp.float32),
            pltpu.VMEM((block_rows, d), jnp.float32),
            pltpu.SemaphoreType.DMA,
            pltpu.SemaphoreType.DMA,
            pltpu.SemaphoreType.DMA,
            pltpu.SemaphoreType.DMA,
        ],
    )
    def k(table_hbm, idx_hbm, out_hbm, idxv, r0, r1, r2, r3, s0, s1, s2, s3):
        wid = lax.axis_index("s") * 2 + lax.axis_index("c")
        base = wid * bpw
        rows = [r0, r1, r2, r3]
        sems = [s0, s1, s2, s3]
        pltpu.sync_copy(idx_hbm.at[wid], idxv)

        def body(j, _):
            gs = []
            for t in range(4):
                i = j * 4 + t
                off = base + i * block_rows

                @pl.when(j > 0)
                def _(t=t, off=off):
                    pltpu.make_async_copy(
                        rows[t], out_hbm.at[pl.ds(off, block_rows)],
                        sems[t]).wait()

                gs.append(pltpu.async_copy(
                    table_hbm.at[idxv.at[i]], rows[t], sems[t]))
            for t in range(4):
                i = j * 4 + t
                off = base + i * block_rows
                gs[t].wait()
                pltpu.async_copy(rows[t], out_hbm.at[pl.ds(off, block_rows)],
                                 sems[t])
            return 0

        lax.fori_loop(0, nblk // 4, body, 0)
        for t in range(4):
            off = base + (nblk - 4 + t) * block_rows
            pltpu.make_async_copy(
                rows[t], out_hbm.at[pl.ds(off, block_rows)], sems[t]).wait()

    return k(table, idx3)


def _sc_scatter_add(rows, idx, n_chunks, block_rows):
    b, d = rows.shape
    dc = d // n_chunks
    bpw = b // NW
    nblk = bpw // block_rows
    assert nblk % 4 == 0 and bpw == nblk * block_rows
    rpt = N_PAD // 16
    zeros_pad = jnp.zeros((N_PAD, dc), jnp.float32)
    idx3 = idx.reshape(NW, nblk, block_rows)

    @functools.partial(
        pl.kernel,
        mesh=plsc.VectorSubcoreMesh(**_MESH),
        out_type=jax.ShapeDtypeStruct((2, N_PAD, d), jnp.float32),
        scratch_types=[
            pltpu.VMEM((nblk, block_rows), jnp.int32),
            pltpu.VMEM((block_rows, dc), jnp.float32),
            pltpu.VMEM((block_rows, dc), jnp.float32),
            pltpu.VMEM((block_rows, dc), jnp.float32),
            pltpu.VMEM((block_rows, dc), jnp.float32),
            pltpu.SemaphoreType.DMA,
            pltpu.SemaphoreType.DMA,
            pltpu.SemaphoreType.DMA,
            pltpu.SemaphoreType.DMA,
            pltpu.SemaphoreType.DMA,
            pltpu.SemaphoreType.DMA,
            pltpu.SemaphoreType.DMA,
            pltpu.SemaphoreType.DMA,
            pltpu.VMEM_SHARED((N_PAD, dc), jnp.float32),
        ],
    )
    def k(rows_hbm, idx_hbm, zero_hbm, out_hbm, idxv, r0, r1, r2, r3,
          s0, s1, s2, s3, a0, a1, a2, a3, acc):
        cid = lax.axis_index("c")
        sid = lax.axis_index("s")
        wid = sid * 2 + cid
        base = wid * bpw
        row0 = sid * rpt
        bufs = [r0, r1, r2, r3]
        sems = [s0, s1, s2, s3]
        asems = [a0, a1, a2, a3]
        pltpu.sync_copy(idx_hbm.at[wid], idxv)

        for ch in range(n_chunks):
            c0 = ch * dc
            pltpu.sync_copy(zero_hbm.at[pl.ds(row0, rpt)],
                            acc.at[pl.ds(row0, rpt)])
            plsc.subcore_barrier()

            def body(j, _):
                gs = []
                for t in range(4):
                    i = j * 4 + t
                    off = base + i * block_rows

                    @pl.when(j > 0)
                    def _(t=t, i=i):
                        pltpu.make_async_copy(bufs[t], acc.at[idxv.at[i]],
                                              asems[t]).wait()

                    if n_chunks == 1:
                        src = rows_hbm.at[pl.ds(off, block_rows)]
                    else:
                        src = rows_hbm.at[pl.ds(off, block_rows),
                                          pl.ds(c0, dc)]
                    gs.append(pltpu.async_copy(src, bufs[t], sems[t]))
                for t in range(4):
                    i = j * 4 + t
                    gs[t].wait()
                    pltpu.async_copy(bufs[t], acc.at[idxv.at[i]], asems[t],
                                     add=True)
                return 0

            lax.fori_loop(0, nblk // 4, body, 0)
            for t in range(4):
                i = nblk - 4 + t
                pltpu.make_async_copy(bufs[t], acc.at[idxv.at[i]],
                                      asems[t]).wait()
            plsc.subcore_barrier()
            if n_chunks == 1:
                pltpu.sync_copy(acc.at[pl.ds(row0, rpt)],
                                out_hbm.at[cid, pl.ds(row0, rpt)])
            else:
                pltpu.sync_copy(
                    acc.at[pl.ds(row0, rpt)],
                    out_hbm.at[cid, pl.ds(row0, rpt), pl.ds(c0, dc)])

    return k(rows, idx3, zeros_pad)


def _proj_body(x_ref, wl_ref, wr_ref, bl_ref, br_ref, xl_ref, xr_ref):
    xv = x_ref[...]
    xl_ref[...] = jnp.dot(xv, wl_ref[...], preferred_element_type=jnp.float32) + bl_ref[...]
    xr_ref[...] = jnp.dot(xv, wr_ref[...], preferred_element_type=jnp.float32) + br_ref[...]


def _proj(xin, wlT, wrT, bl, br, bn):
    n, kdim = xin.shape
    d = wlT.shape[1]
    out = jax.ShapeDtypeStruct((n, d), jnp.float32)
    return pl.pallas_call(
        _proj_body,
        grid=(n // bn,),
        in_specs=[
            pl.BlockSpec((bn, kdim), lambda i: (i, 0)),
            pl.BlockSpec((kdim, d), lambda i: (0, 0)),
            pl.BlockSpec((kdim, d), lambda i: (0, 0)),
            pl.BlockSpec((1, d), lambda i: (0, 0)),
            pl.BlockSpec((1, d), lambda i: (0, 0)),
        ],
        out_specs=[
            pl.BlockSpec((bn, d), lambda i: (i, 0)),
            pl.BlockSpec((bn, d), lambda i: (i, 0)),
        ],
        out_shape=[out, out],
    )(xin, wlT, wrT, bl.reshape(1, d), br.reshape(1, d))


def _gat_proj_body(p_ref, dp_ref, bsel_ref, bias_ref, wl_ref, wr_ref, bl_ref,
                   br_ref, xl_ref, xr_ref):
    den = jnp.dot(dp_ref[0] + dp_ref[1], bsel_ref[...],
                  preferred_element_type=jnp.float32)
    den = jnp.maximum(den, 1e-30)
    h = jnp.tanh((p_ref[0] + p_ref[1]) / den + bias_ref[...])
    xl_ref[...] = jnp.dot(h, wl_ref[...], preferred_element_type=jnp.float32) + bl_ref[...]
    xr_ref[...] = jnp.dot(h, wr_ref[...], preferred_element_type=jnp.float32) + br_ref[...]


def _gat_proj(parts, denp, bsel, bias, wlT, wrT, bl, br, bn):
    _, n, kdim = parts.shape
    d = wlT.shape[1]
    out = jax.ShapeDtypeStruct((n, d), jnp.float32)
    return pl.pallas_call(
        _gat_proj_body,
        grid=(n // bn,),
        in_specs=[
            pl.BlockSpec((2, bn, kdim), lambda i: (0, i, 0)),
            pl.BlockSpec((2, bn, 128), lambda i: (0, i, 0)),
            pl.BlockSpec((128, kdim), lambda i: (0, 0)),
            pl.BlockSpec((1, kdim), lambda i: (0, 0)),
            pl.BlockSpec((kdim, d), lambda i: (0, 0)),
            pl.BlockSpec((kdim, d), lambda i: (0, 0)),
            pl.BlockSpec((1, d), lambda i: (0, 0)),
            pl.BlockSpec((1, d), lambda i: (0, 0)),
        ],
        out_specs=[
            pl.BlockSpec((bn, d), lambda i: (i, 0)),
            pl.BlockSpec((bn, d), lambda i: (i, 0)),
        ],
        out_shape=[out, out],
    )(parts, denp, bsel, bias.reshape(1, kdim), wlT, wrT,
      bl.reshape(1, d), br.reshape(1, d))


def _alpha_body(xlg_ref, xrg_ref, ea_ref, we_ref, att_ref, sel_ref, o_ref):
    u = xlg_ref[...] + xrg_ref[...] + jnp.dot(
        ea_ref[...], we_ref[...], preferred_element_type=jnp.float32)
    m = jnp.maximum(u, 0.2 * u)
    o_ref[...] = jnp.dot(m * att_ref[...], sel_ref[...],
                         preferred_element_type=jnp.float32)


def _alpha(xlg, xrg, ea, weT, att_row, sel, be):
    b, d = xlg.shape
    return pl.pallas_call(
        _alpha_body,
        grid=(b // be,),
        in_specs=[
            pl.BlockSpec((be, d), lambda i: (i, 0)),
            pl.BlockSpec((be, d), lambda i: (i, 0)),
            pl.BlockSpec((be, 16), lambda i: (i, 0)),
            pl.BlockSpec((16, d), lambda i: (0, 0)),
            pl.BlockSpec((1, d), lambda i: (0, 0)),
            pl.BlockSpec((d, 128), lambda i: (0, 0)),
        ],
        out_specs=pl.BlockSpec((be, 128), lambda i: (i, 0)),
        out_shape=jax.ShapeDtypeStruct((b, 128), jnp.float32),
    )(xlg, xrg, ea, weT, att_row, sel)


def _ew_body(xlg_ref, a_ref, g_ref, bsel_ref, w_ref, ex_ref, *, be, n_valid):
    gid = pl.program_id(0) * be + lax.broadcasted_iota(jnp.int32, (be, 128), 0)
    lane = lax.broadcasted_iota(jnp.int32, (be, 128), 1)
    ok = (gid < n_valid) & (lane < 8)
    ex = jnp.where(ok, jnp.exp(a_ref[...] - g_ref[0, 0]), 0.0)
    ex_ref[...] = ex
    w_ref[...] = jnp.dot(ex, bsel_ref[...],
                         preferred_element_type=jnp.float32) * xlg_ref[...]


def _exp_weight(xlg, alpha, gmax, bsel, be):
    b, d = xlg.shape
    return pl.pallas_call(
        functools.partial(_ew_body, be=be, n_valid=EP),
        grid=(b // be,),
        in_specs=[
            pl.BlockSpec((be, d), lambda i: (i, 0)),
            pl.BlockSpec((be, 128), lambda i: (i, 0)),
            pl.BlockSpec((1, 1), lambda i: (0, 0)),
            pl.BlockSpec((128, d), lambda i: (0, 0)),
        ],
        out_specs=[
            pl.BlockSpec((be, d), lambda i: (i, 0)),
            pl.BlockSpec((be, 128), lambda i: (i, 0)),
        ],
        out_shape=[jax.ShapeDtypeStruct((b, d), jnp.float32),
                   jax.ShapeDtypeStruct((b, 128), jnp.float32)],
    )(xlg, alpha, gmax.reshape(1, 1), bsel)


def _head_body(p_ref, dp_ref, bsel_ref, bias2_ref, wih_ref, b_ref, wfc_ref,
               bfc_ref, o_ref):
    den = jnp.dot(dp_ref[0] + dp_ref[1], bsel_ref[...],
                  preferred_element_type=jnp.float32)
    s = (p_ref[0] + p_ref[1]) / jnp.maximum(den, 1e-30)
    mean = s[:, 0:128]
    for h in range(1, 8):
        mean = mean + s[:, h * 128:(h + 1) * 128]
    h2 = jnp.tanh(mean * 0.125 + bias2_ref[...])
    gates = jnp.dot(h2, wih_ref[...], preferred_element_type=jnp.float32) + b_ref[...]
    i, f, g, o = jnp.split(gates, 4, axis=1)
    c = jax.nn.sigmoid(i) * jnp.tanh(g)
    hd = jax.nn.sigmoid(o) * jnp.tanh(c)
    o_ref[...] = jnp.sum(hd * wfc_ref[...], axis=1, keepdims=True) + bfc_ref[0, 0]


def _head(parts, denp, bsel, bias2, WihT, b, Wfc, bfc, bn):
    _, n, d = parts.shape
    return pl.pallas_call(
        _head_body,
        grid=(n // bn,),
        in_specs=[
            pl.BlockSpec((2, bn, d), lambda i: (0, i, 0)),
            pl.BlockSpec((2, bn, 128), lambda i: (0, i, 0)),
            pl.BlockSpec((128, d), lambda i: (0, 0)),
            pl.BlockSpec((1, 128), lambda i: (0, 0)),
            pl.BlockSpec((128, 128), lambda i: (0, 0)),
            pl.BlockSpec((1, 128), lambda i: (0, 0)),
            pl.BlockSpec((1, 32), lambda i: (0, 0)),
            pl.BlockSpec((1, 1), lambda i: (0, 0)),
        ],
        out_specs=pl.BlockSpec((bn, 1), lambda i: (i, 0)),
        out_shape=jax.ShapeDtypeStruct((n, 1), jnp.float32),
    )(parts, denp, bsel, bias2.reshape(1, 128), WihT, b.reshape(1, 128),
      Wfc, bfc.reshape(1, 1))


def _selectors(d, c_per_head):
    ch = jnp.arange(d, dtype=jnp.int32) // c_per_head
    hh = jnp.arange(128, dtype=jnp.int32)
    sel = ((ch[:, None] == hh[None, :]) & (hh[None, :] < 8)).astype(jnp.float32)
    return sel, sel.T


def _pad_rows(a, n):
    return jnp.pad(a, ((0, n - a.shape[0]),) + ((0, 0),) * (a.ndim - 1))


def kernel(x, edge_index, edge_attr, Wl1, bl1, Wr1, br1, We1, att1, bias1,
           Wl2, bl2, Wr2, br2, We2, att2, bias2, Wih, Whh, bih, bhh, Wfc, bfc):
    n = x.shape[0]
    src0 = edge_index[0].astype(jnp.int32)
    dst0 = edge_index[1].astype(jnp.int32)
    ar = jnp.arange(n, dtype=jnp.int32)
    src_p = _pad_rows(jnp.concatenate([src0, ar])[:, None], EP_PAD)[:, 0]
    dst_p = _pad_rows(jnp.concatenate([dst0, ar])[:, None], EP_PAD)[:, 0]

    ea_ext = jnp.concatenate(
        [edge_attr, jnp.ones((N_EDGES, 1), jnp.float32),
         jnp.zeros((N_EDGES, 111), jnp.float32)], axis=1)
    s0 = _sc_scatter_add(_pad_rows(ea_ext, E0_PAD),
                         _pad_rows(dst0[:, None], E0_PAD)[:, 0],
                         n_chunks=1, block_rows=64)
    ssum = s0[0] + s0[1]
    cnt = ssum[:n, 16:17]
    mean_attr = ssum[:n, :16] / jnp.maximum(cnt, 1.0)
    ea_full = _pad_rows(jnp.concatenate([edge_attr, mean_attr], axis=0), EP_PAD)

    x_p = _pad_rows(x, N_PAD)

    sel1, bsel1 = _selectors(512, 64)
    xl1, xr1 = _proj(x_p, Wl1.T, Wr1.T, bl1, br1, bn=632)
    xlg1 = _sc_gather(xl1, src_p, block_rows=48)
    xrg1 = _sc_gather(xr1, dst_p, block_rows=48)
    alpha1 = _alpha(xlg1, xrg1, ea_full, We1.T, att1.reshape(1, 512), sel1, be=2048)
    gmax1 = jnp.max(alpha1[:, :8])
    w1, ex1 = _exp_weight(xlg1, alpha1, gmax1, bsel1, be=2048)
    den1p = _sc_scatter_add(ex1, dst_p, n_chunks=1, block_rows=64)
    out1p = _sc_scatter_add(w1, dst_p, n_chunks=4, block_rows=64)

    sel2, bsel2 = _selectors(1024, 128)
    xl2, xr2 = _gat_proj(out1p, den1p, bsel1, bias1, Wl2.T, Wr2.T, bl2, br2,
                         bn=632)
    xlg2 = _sc_gather(xl2, src_p, block_rows=24)
    xrg2 = _sc_gather(xr2, dst_p, block_rows=24)
    alpha2 = _alpha(xlg2, xrg2, ea_full, We2.T, att2.reshape(1, 1024), sel2, be=1024)
    gmax2 = jnp.max(alpha2[:, :8])
    w2, ex2 = _exp_weight(xlg2, alpha2, gmax2, bsel2, be=1024)
    den2p = _sc_scatter_add(ex2, dst_p, n_chunks=1, block_rows=64)
    out2p = _sc_scatter_add(w2, dst_p, n_chunks=8, block_rows=64)

    y = _head(out2p, den2p, bsel2, bias2, Wih.T, bih + bhh, Wfc, bfc, bn=632)
    return y[:n]
